# Initial kernel scaffold; baseline (speedup 1.0000x reference)
import jax
import jax.numpy as jnp
from jax.experimental import pallas as pl


def kernel(x, edge_index, W, b, ln_w, ln_b):
    raise NotImplementedError("write your pallas kernel here")


# speedup vs baseline: 12.1618x; 12.1618x over previous
import functools

import jax
import jax.numpy as jnp
from jax import lax
from jax.experimental import pallas as pl
from jax.experimental.pallas import tpu as pltpu
from jax.experimental.pallas import tpu_sc as plsc

CH = 128


def _sc_mesh():
    return plsc.VectorSubcoreMesh(core_axis_name="c", subcore_axis_name="s")


def _make_deg_kernel(NP, EP):
    n_per_tile = EP // 32
    nch = n_per_tile // CH
    slc = NP // 16

    @functools.partial(
        pl.kernel,
        mesh=_sc_mesh(),
        out_type=jax.ShapeDtypeStruct((2 * NP,), jnp.float32),
        scratch_types=[
            pltpu.VMEM_SHARED((NP,), jnp.float32),
            pltpu.VMEM((CH,), jnp.float32),
            pltpu.VMEM((CH,), jnp.int32),
            pltpu.VMEM((slc,), jnp.float32),
        ],
    )
    def deg_k(col_hbm, deg_out, deg_sh, ones_v, cidx_v, zb_v):
        c = lax.axis_index("c")
        s = lax.axis_index("s")

        def fill_ones(i, _):
            ones_v[pl.ds(i * 16, 16)] = jnp.full((16,), 1.0, jnp.float32)
            return 0

        lax.fori_loop(0, CH // 16, fill_ones, 0)

        def fill_z(i, _):
            zb_v[pl.ds(i * 16, 16)] = jnp.zeros((16,), jnp.float32)
            return 0

        lax.fori_loop(0, slc // 16, fill_z, 0)
        pltpu.sync_copy(zb_v, deg_sh.at[pl.ds(s * slc, slc)])
        plsc.subcore_barrier()

        base = c * (EP // 2) + s * n_per_tile

        def step(j, _):
            pltpu.sync_copy(col_hbm.at[pl.ds(base + j * CH, CH)], cidx_v)
            pltpu.sync_copy(ones_v, deg_sh.at[cidx_v], add=True)
            return 0

        lax.fori_loop(0, nch, step, 0)
        plsc.subcore_barrier()
        pltpu.sync_copy(
            deg_sh.at[pl.ds(s * slc, slc)],
            deg_out.at[pl.ds(c * NP + s * slc, slc)],
        )

    return deg_k


def _make_agg_kernel(NP, EP, H):
    n_per_tile = EP // 16
    nch = n_per_tile // CH
    slc = NP // 16

    @functools.partial(
        pl.kernel,
        mesh=_sc_mesh(),
        out_type=jax.ShapeDtypeStruct((2 * NP, H), jnp.float32),
        scratch_types=[
            pltpu.VMEM_SHARED((NP, H), jnp.float32),
            pltpu.VMEM((CH, H), jnp.float32),
            pltpu.VMEM((CH,), jnp.int32),
            pltpu.VMEM((CH,), jnp.int32),
            pltpu.SemaphoreType.DMA,
        ],
    )
    def agg_k(y_hbm, row_hbm, col_hbm, out_hbm, agg_sh, rows_v, ridx_v, cidx_v, sem):
        c = lax.axis_index("c")
        s = lax.axis_index("s")

        def fz(i, _):
            r = i // (H // 16)
            k = (i % (H // 16)) * 16
            rows_v[r, pl.ds(k, 16)] = jnp.zeros((16,), jnp.float32)
            return 0

        lax.fori_loop(0, (CH * H) // 16, fz, 0)

        def zcp(k, _):
            pltpu.sync_copy(rows_v, agg_sh.at[pl.ds(s * slc + k * CH, CH)])
            return 0

        lax.fori_loop(0, slc // CH, zcp, 0)
        plsc.subcore_barrier()

        rbase = c * EP + s * n_per_tile
        cbase = s * n_per_tile

        def step(j, _):
            pltpu.sync_copy(row_hbm.at[pl.ds(rbase + j * CH, CH)], ridx_v)
            pltpu.sync_copy(col_hbm.at[pl.ds(cbase + j * CH, CH)], cidx_v)
            pltpu.async_copy(y_hbm.at[ridx_v], rows_v, sem).wait()
            pltpu.sync_copy(rows_v, agg_sh.at[cidx_v], add=True)
            return 0

        lax.fori_loop(0, nch, step, 0)
        plsc.subcore_barrier()
        pltpu.sync_copy(
            agg_sh.at[pl.ds(s * slc, slc)],
            out_hbm.at[pl.ds(c * NP + s * slc, slc)],
        )

    return agg_k


def _make_pre_kernel(NP, D, BN):
    H = D // 2

    def body(x_ref, w_ref, degs_ref, y2_ref, dis_ref):
        deg = degs_ref[0] + degs_ref[1] + 1.0
        dis = lax.rsqrt(deg)
        xw = jnp.dot(x_ref[...], w_ref[...], preferred_element_type=jnp.float32)
        y = xw * dis[:, None]
        y2_ref[0] = y[:, :H]
        y2_ref[1] = y[:, H:]
        dis_ref[...] = dis

    return pl.pallas_call(
        body,
        grid=(NP // BN,),
        in_specs=[
            pl.BlockSpec((BN, D), lambda i: (i, 0)),
            pl.BlockSpec((D, D), lambda i: (0, 0)),
            pl.BlockSpec((2, BN), lambda i: (0, i)),
        ],
        out_specs=[
            pl.BlockSpec((2, BN, H), lambda i: (0, i, 0)),
            pl.BlockSpec((BN,), lambda i: (i,)),
        ],
        out_shape=[
            jax.ShapeDtypeStruct((2, NP, H), jnp.float32),
            jax.ShapeDtypeStruct((NP,), jnp.float32),
        ],
    )


def _make_post_kernel(NP, D, BN):
    H = D // 2

    def body(agg_ref, y2_ref, dis_ref, x_ref, b_ref, lw_ref, lb_ref, o_ref):
        agg = jnp.concatenate([agg_ref[0], agg_ref[1]], axis=1)
        y = jnp.concatenate([y2_ref[0], y2_ref[1]], axis=1)
        dis = dis_ref[...]
        h = (agg + y) * dis[:, None] + b_ref[...][None, :]
        mu = jnp.mean(h, axis=1, keepdims=True)
        d = h - mu
        var = jnp.mean(d * d, axis=1, keepdims=True)
        h = d * lax.rsqrt(var + 1e-5) * lw_ref[...][None, :] + lb_ref[...][None, :]
        h = jnp.maximum(h, 0.0)
        o_ref[...] = h + x_ref[...]

    return pl.pallas_call(
        body,
        grid=(NP // BN,),
        in_specs=[
            pl.BlockSpec((2, BN, H), lambda i: (0, i, 0)),
            pl.BlockSpec((2, BN, H), lambda i: (0, i, 0)),
            pl.BlockSpec((BN,), lambda i: (i,)),
            pl.BlockSpec((BN, D), lambda i: (i, 0)),
            pl.BlockSpec((D,), lambda i: (0,)),
            pl.BlockSpec((D,), lambda i: (0,)),
            pl.BlockSpec((D,), lambda i: (0,)),
        ],
        out_specs=pl.BlockSpec((BN, D), lambda i: (i, 0)),
        out_shape=jax.ShapeDtypeStruct((NP, D), jnp.float32),
    )


def kernel(x, edge_index, W, b, ln_w, ln_b):
    N, D = x.shape
    E = edge_index.shape[1]
    H = D // 2
    NP = ((N + 2047) // 2048) * 2048
    EP = ((E + 4095) // 4096) * 4096
    BN = 512

    row = edge_index[0]
    col = edge_index[1]
    pad = EP - E
    if pad:
        fill = (N + (jnp.arange(pad, dtype=jnp.int32) % (NP - N))).astype(jnp.int32)
        row = jnp.concatenate([row, fill])
        col = jnp.concatenate([col, fill])
    row2 = jnp.concatenate([row, row + NP])

    x_pad = jnp.zeros((NP, D), jnp.float32).at[:N].set(x)

    degs = _make_deg_kernel(NP, EP)(col)
    y2, dis = _make_pre_kernel(NP, D, BN)(x_pad, W, degs.reshape(2, NP))
    agg = _make_agg_kernel(NP, EP, H)(y2.reshape(2 * NP, H), row2, col)
    out = _make_post_kernel(NP, D, BN)(
        agg.reshape(2, NP, H), y2, dis, x_pad, b, ln_w, ln_b
    )
    return out[:N]


# speedup vs baseline: 19.6180x; 1.6131x over previous
import functools

import jax
import jax.numpy as jnp
from jax import lax
from jax.experimental import pallas as pl
from jax.experimental.pallas import tpu as pltpu
from jax.experimental.pallas import tpu_sc as plsc

CH = 128


def _sc_mesh():
    return plsc.VectorSubcoreMesh(core_axis_name="c", subcore_axis_name="s")


def _make_deg_kernel(NP, EP):
    n_per_tile = EP // 32
    nch = n_per_tile // CH
    slc = NP // 16

    @functools.partial(
        pl.kernel,
        mesh=_sc_mesh(),
        out_type=jax.ShapeDtypeStruct((2 * NP,), jnp.float32),
        scratch_types=[
            pltpu.VMEM_SHARED((NP,), jnp.float32),
            pltpu.VMEM((CH,), jnp.float32),
            pltpu.VMEM((nch, CH), jnp.int32),
            pltpu.VMEM((slc,), jnp.float32),
            pltpu.SemaphoreType.DMA,
        ],
    )
    def deg_k(col2d_hbm, deg_out, deg_sh, ones_v, cidx_all, zb_v, sem):
        c = lax.axis_index("c")
        s = lax.axis_index("s")
        tch = c * (EP // 2 // CH) + s * nch

        def fill_ones(i, _):
            ones_v[pl.ds(i * 16, 16)] = jnp.full((16,), 1.0, jnp.float32)
            return 0

        lax.fori_loop(0, CH // 16, fill_ones, 0)

        def fill_z(i, _):
            zb_v[pl.ds(i * 16, 16)] = jnp.zeros((16,), jnp.float32)
            return 0

        lax.fori_loop(0, slc // 16, fill_z, 0)
        pltpu.sync_copy(col2d_hbm.at[pl.ds(tch, nch)], cidx_all)
        pltpu.sync_copy(zb_v, deg_sh.at[pl.ds(s * slc, slc)])
        plsc.subcore_barrier()

        def fire(j, _):
            pltpu.async_copy(ones_v, deg_sh.at[cidx_all.at[j]], sem, add=True)
            return 0

        lax.fori_loop(0, nch, fire, 0)

        def drain(j, _):
            pltpu.make_async_copy(ones_v, deg_sh.at[cidx_all.at[j]], sem).wait()
            return 0

        lax.fori_loop(0, nch, drain, 0)
        plsc.subcore_barrier()
        pltpu.sync_copy(
            deg_sh.at[pl.ds(s * slc, slc)],
            deg_out.at[pl.ds(c * NP + s * slc, slc)],
        )

    return deg_k


def _make_agg_kernel(NP, EP, H):
    n_per_tile = EP // 16
    nch = n_per_tile // CH
    slc = NP // 16

    @functools.partial(
        pl.kernel,
        mesh=_sc_mesh(),
        out_type=jax.ShapeDtypeStruct((2 * NP, H), jnp.float32),
        scratch_types=[
            pltpu.VMEM_SHARED((NP, H), jnp.float32),
            pltpu.VMEM((CH, H), jnp.float32),
            pltpu.VMEM((CH, H), jnp.float32),
            pltpu.VMEM((2, CH), jnp.int32),
            pltpu.VMEM((2, CH), jnp.int32),
            pltpu.SemaphoreType.DMA,
            pltpu.SemaphoreType.DMA,
            pltpu.SemaphoreType.DMA,
            pltpu.SemaphoreType.DMA,
        ],
    )
    def agg_k(y_hbm, rc_hbm, out_hbm, agg_sh, buf0, buf1, rc0, rc1,
              sem0, sem1, semr0, semr1):
        c = lax.axis_index("c")
        s = lax.axis_index("s")
        tbase = 2 * (c * (EP // CH) + s * nch)

        def fz(i, _):
            r = i // (H // 16)
            k = (i % (H // 16)) * 16
            buf0[r, pl.ds(k, 16)] = jnp.zeros((16,), jnp.float32)
            return 0

        lax.fori_loop(0, (CH * H) // 16, fz, 0)

        def zcp(k, _):
            pltpu.sync_copy(buf0, agg_sh.at[pl.ds(s * slc + k * CH, CH)])
            return 0

        lax.fori_loop(0, slc // CH, zcp, 0)

        def rstart(j, rc, semr):
            pltpu.async_copy(rc_hbm.at[pl.ds(tbase + 2 * j, 2)], rc, semr)

        def rwait(j, rc, semr):
            pltpu.make_async_copy(
                rc_hbm.at[pl.ds(tbase + 2 * j, 2)], rc, semr
            ).wait()

        def gstart(rc, buf, sem):
            pltpu.async_copy(y_hbm.at[rc.at[0]], buf, sem)

        def gwait(rc, buf, sem):
            pltpu.make_async_copy(y_hbm.at[rc.at[0]], buf, sem).wait()

        rstart(0, rc0, semr0)
        plsc.subcore_barrier()
        rwait(0, rc0, semr0)
        gstart(rc0, buf0, sem0)
        rstart(1, rc1, semr1)

        def step(i, _):
            j1 = 2 * i + 1
            j2 = 2 * i + 2
            j3 = 2 * i + 3
            gwait(rc0, buf0, sem0)
            rwait(j1, rc1, semr1)
            gstart(rc1, buf1, sem1)
            pltpu.sync_copy(buf0, agg_sh.at[rc0.at[1]], add=True)

            @pl.when(j2 < nch)
            def _():
                rstart(j2, rc0, semr0)

            gwait(rc1, buf1, sem1)

            @pl.when(j2 < nch)
            def _():
                rwait(j2, rc0, semr0)
                gstart(rc0, buf0, sem0)

            pltpu.sync_copy(buf1, agg_sh.at[rc1.at[1]], add=True)

            @pl.when(j3 < nch)
            def _():
                rstart(j3, rc1, semr1)

            return 0

        lax.fori_loop(0, nch // 2, step, 0)
        plsc.subcore_barrier()
        pltpu.sync_copy(
            agg_sh.at[pl.ds(s * slc, slc)],
            out_hbm.at[pl.ds(c * NP + s * slc, slc)],
        )

    return agg_k


def _make_pre_kernel(NP, D, BN):
    H = D // 2

    def body(x_ref, w_ref, degs_ref, y2_ref, dis_ref):
        deg = degs_ref[0] + degs_ref[1] + 1.0
        dis = lax.rsqrt(deg)
        xw = jnp.dot(x_ref[...], w_ref[...], preferred_element_type=jnp.float32)
        y = xw * dis[:, None]
        y2_ref[0] = y[:, :H]
        y2_ref[1] = y[:, H:]
        dis_ref[...] = dis

    return pl.pallas_call(
        body,
        grid=(NP // BN,),
        in_specs=[
            pl.BlockSpec((BN, D), lambda i: (i, 0)),
            pl.BlockSpec((D, D), lambda i: (0, 0)),
            pl.BlockSpec((2, BN), lambda i: (0, i)),
        ],
        out_specs=[
            pl.BlockSpec((2, BN, H), lambda i: (0, i, 0)),
            pl.BlockSpec((BN,), lambda i: (i,)),
        ],
        out_shape=[
            jax.ShapeDtypeStruct((2, NP, H), jnp.float32),
            jax.ShapeDtypeStruct((NP,), jnp.float32),
        ],
    )


def _make_post_kernel(NP, D, BN):
    H = D // 2

    def body(agg_ref, y2_ref, dis_ref, x_ref, b_ref, lw_ref, lb_ref, o_ref):
        agg = jnp.concatenate([agg_ref[0], agg_ref[1]], axis=1)
        y = jnp.concatenate([y2_ref[0], y2_ref[1]], axis=1)
        dis = dis_ref[...]
        h = (agg + y) * dis[:, None] + b_ref[...][None, :]
        mu = jnp.mean(h, axis=1, keepdims=True)
        d = h - mu
        var = jnp.mean(d * d, axis=1, keepdims=True)
        h = d * lax.rsqrt(var + 1e-5) * lw_ref[...][None, :] + lb_ref[...][None, :]
        h = jnp.maximum(h, 0.0)
        o_ref[...] = h + x_ref[...]

    return pl.pallas_call(
        body,
        grid=(NP // BN,),
        in_specs=[
            pl.BlockSpec((2, BN, H), lambda i: (0, i, 0)),
            pl.BlockSpec((2, BN, H), lambda i: (0, i, 0)),
            pl.BlockSpec((BN,), lambda i: (i,)),
            pl.BlockSpec((BN, D), lambda i: (i, 0)),
            pl.BlockSpec((D,), lambda i: (0,)),
            pl.BlockSpec((D,), lambda i: (0,)),
            pl.BlockSpec((D,), lambda i: (0,)),
        ],
        out_specs=pl.BlockSpec((BN, D), lambda i: (i, 0)),
        out_shape=jax.ShapeDtypeStruct((NP, D), jnp.float32),
    )


def kernel(x, edge_index, W, b, ln_w, ln_b):
    N, D = x.shape
    E = edge_index.shape[1]
    H = D // 2
    NP = ((N + 2047) // 2048) * 2048
    EP = ((E + 4095) // 4096) * 4096
    BN = 512

    row = edge_index[0]
    col = edge_index[1]
    pad = EP - E
    if pad:
        fill = (N + (jnp.arange(pad, dtype=jnp.int32) % (NP - N))).astype(jnp.int32)
        row = jnp.concatenate([row, fill])
        col = jnp.concatenate([col, fill])
    row2 = jnp.concatenate([row, row + NP])

    x_pad = jnp.zeros((NP, D), jnp.float32).at[:N].set(x)

    col2d = col.reshape(EP // CH, CH)
    rowc = row2.reshape(2, EP // CH, CH)
    colc = jnp.broadcast_to(col2d[None], (2, EP // CH, CH))
    rc = jnp.stack([rowc, colc], axis=2).reshape(4 * (EP // CH), CH)
    degs = _make_deg_kernel(NP, EP)(col2d)
    y2, dis = _make_pre_kernel(NP, D, BN)(x_pad, W, degs.reshape(2, NP))
    agg = _make_agg_kernel(NP, EP, H)(y2.reshape(2 * NP, H), rc)
    out = _make_post_kernel(NP, D, BN)(
        agg.reshape(2, NP, H), y2, dis, x_pad, b, ln_w, ln_b
    )
    return out[:N]


# speedup vs baseline: 21.2077x; 1.0810x over previous
import functools

import jax
import jax.numpy as jnp
from jax import lax
from jax.experimental import pallas as pl
from jax.experimental.pallas import tpu as pltpu
from jax.experimental.pallas import tpu_sc as plsc

CH = 128


def _sc_mesh():
    return plsc.VectorSubcoreMesh(core_axis_name="c", subcore_axis_name="s")


def _make_deg_kernel(NP, EP):
    n_per_tile = EP // 32
    nch = n_per_tile // CH
    slc = NP // 16

    @functools.partial(
        pl.kernel,
        mesh=_sc_mesh(),
        out_type=jax.ShapeDtypeStruct((2 * NP,), jnp.float32),
        scratch_types=[
            pltpu.VMEM_SHARED((NP,), jnp.float32),
            pltpu.VMEM((CH,), jnp.float32),
            pltpu.VMEM((nch, CH), jnp.int32),
            pltpu.VMEM((slc,), jnp.float32),
            pltpu.SemaphoreType.DMA,
        ],
    )
    def deg_k(col2d_hbm, deg_out, deg_sh, ones_v, cidx_all, zb_v, sem):
        c = lax.axis_index("c")
        s = lax.axis_index("s")
        tch = c * (EP // 2 // CH) + s * nch

        def fill_ones(i, _):
            ones_v[pl.ds(i * 16, 16)] = jnp.full((16,), 1.0, jnp.float32)
            return 0

        lax.fori_loop(0, CH // 16, fill_ones, 0)

        def fill_z(i, _):
            zb_v[pl.ds(i * 16, 16)] = jnp.zeros((16,), jnp.float32)
            return 0

        lax.fori_loop(0, slc // 16, fill_z, 0)
        pltpu.sync_copy(col2d_hbm.at[pl.ds(tch, nch)], cidx_all)
        pltpu.sync_copy(zb_v, deg_sh.at[pl.ds(s * slc, slc)])
        plsc.subcore_barrier()

        def fire(j, _):
            pltpu.async_copy(ones_v, deg_sh.at[cidx_all.at[j]], sem, add=True)
            return 0

        lax.fori_loop(0, nch, fire, 0)

        def drain(j, _):
            pltpu.make_async_copy(ones_v, deg_sh.at[cidx_all.at[j]], sem).wait()
            return 0

        lax.fori_loop(0, nch, drain, 0)
        plsc.subcore_barrier()
        pltpu.sync_copy(
            deg_sh.at[pl.ds(s * slc, slc)],
            deg_out.at[pl.ds(c * NP + s * slc, slc)],
        )

    return deg_k


def _make_agg_kernel(NP, EP, H):
    n_per_tile = EP // 16
    nch = n_per_tile // CH
    slc = NP // 16

    assert nch % 4 == 0

    @functools.partial(
        pl.kernel,
        mesh=_sc_mesh(),
        out_type=jax.ShapeDtypeStruct((2 * NP, H), jnp.float32),
        scratch_types=[
            pltpu.VMEM_SHARED((NP, H), jnp.float32),
            pltpu.VMEM((CH, H), jnp.float32),
            pltpu.VMEM((CH, H), jnp.float32),
            pltpu.VMEM((2, CH), jnp.int32),
            pltpu.VMEM((2, CH), jnp.int32),
            pltpu.VMEM((2, CH), jnp.int32),
            pltpu.VMEM((2, CH), jnp.int32),
            pltpu.SemaphoreType.DMA,
            pltpu.SemaphoreType.DMA,
            pltpu.SemaphoreType.DMA,
            pltpu.SemaphoreType.DMA,
            pltpu.SemaphoreType.DMA,
            pltpu.SemaphoreType.DMA,
            pltpu.SemaphoreType.DMA,
            pltpu.SemaphoreType.DMA,
        ],
    )
    def agg_k(y_hbm, rc_hbm, out_hbm, agg_sh, bufa, bufb, r0, r1, r2, r3,
              semga, semgb, semsa, semsb, semr0, semr1, semr2, semr3):
        c = lax.axis_index("c")
        s = lax.axis_index("s")
        tbase = 2 * (c * (EP // CH) + s * nch)

        def rstart(j, rc, semr):
            pltpu.async_copy(rc_hbm.at[pl.ds(tbase + 2 * j, 2)], rc, semr)

        def rwait(j, rc, semr):
            pltpu.make_async_copy(
                rc_hbm.at[pl.ds(tbase + 2 * j, 2)], rc, semr
            ).wait()

        def gstart(rc, buf, sem):
            pltpu.async_copy(y_hbm.at[rc.at[0]], buf, sem)

        def gwait(rc, buf, sem):
            pltpu.make_async_copy(y_hbm.at[rc.at[0]], buf, sem).wait()

        def sstart(buf, rc, sem):
            pltpu.async_copy(buf, agg_sh.at[rc.at[1]], sem, add=True)

        def swait(buf, rc, sem):
            pltpu.make_async_copy(buf, agg_sh.at[rc.at[1]], sem).wait()

        rstart(0, r0, semr0)
        rstart(1, r1, semr1)
        rstart(2, r2, semr2)

        def fz(i, _):
            r = i // (H // 16)
            k = (i % (H // 16)) * 16
            bufa[r, pl.ds(k, 16)] = jnp.zeros((16,), jnp.float32)
            return 0

        lax.fori_loop(0, (CH * H) // 16, fz, 0)

        def zcp(k, _):
            pltpu.sync_copy(bufa, agg_sh.at[pl.ds(s * slc + k * CH, CH)])
            return 0

        lax.fori_loop(0, slc // CH, zcp, 0)
        rwait(0, r0, semr0)
        gstart(r0, bufa, semga)
        plsc.subcore_barrier()

        def sub(j, bj, sgj, ssj, bp, ssp, sgp, rcj, rcn, rcp, semrn, semrp,
                first, guard_pref, guard_g):
            gwait(rcj, bj, sgj)
            sstart(bj, rcj, ssj)

            def after_prev():
                swait(bp, rcp, ssp)

            if first:
                @pl.when(j > 0)
                def _():
                    after_prev()
            else:
                after_prev()

            if guard_pref:
                @pl.when(j + 3 < nch)
                def _():
                    rstart(j + 3, rcp, semrp)
            else:
                rstart(j + 3, rcp, semrp)

            def next_gather():
                rwait(j + 1, rcn, semrn)
                gstart(rcn, bp, sgp)

            if guard_g:
                @pl.when(j + 1 < nch)
                def _():
                    next_gather()
            else:
                next_gather()

        def step(i, _):
            j0 = 4 * i
            sub(j0, bufa, semga, semsa, bufb, semsb, semgb,
                r0, r1, r3, semr1, semr3, True, False, False)
            sub(j0 + 1, bufb, semgb, semsb, bufa, semsa, semga,
                r1, r2, r0, semr2, semr0, False, True, False)
            sub(j0 + 2, bufa, semga, semsa, bufb, semsb, semgb,
                r2, r3, r1, semr3, semr1, False, True, False)
            sub(j0 + 3, bufb, semgb, semsb, bufa, semsa, semga,
                r3, r0, r2, semr0, semr2, False, True, True)
            return 0

        lax.fori_loop(0, nch // 4, step, 0)
        swait(bufb, r3, semsb)
        plsc.subcore_barrier()
        pltpu.sync_copy(
            agg_sh.at[pl.ds(s * slc, slc)],
            out_hbm.at[pl.ds(c * NP + s * slc, slc)],
        )

    return agg_k


def _make_pre_kernel(N, NP, D, BN):
    H = D // 2

    def body(x_ref, w_ref, degs_ref, y2_ref, dis_ref):
        deg = degs_ref[0] + degs_ref[1] + 1.0
        dis = lax.rsqrt(deg)
        xw = jnp.dot(x_ref[...], w_ref[...], preferred_element_type=jnp.float32)
        y = xw * dis[:, None]
        y2_ref[0] = y[:, :H]
        y2_ref[1] = y[:, H:]
        dis_ref[...] = dis

    return pl.pallas_call(
        body,
        grid=(NP // BN,),
        in_specs=[
            pl.BlockSpec((BN, D), lambda i: (i, 0)),
            pl.BlockSpec((D, D), lambda i: (0, 0)),
            pl.BlockSpec((2, BN), lambda i: (0, i)),
        ],
        out_specs=[
            pl.BlockSpec((2, BN, H), lambda i: (0, i, 0)),
            pl.BlockSpec((BN,), lambda i: (i,)),
        ],
        out_shape=[
            jax.ShapeDtypeStruct((2, NP, H), jnp.float32),
            jax.ShapeDtypeStruct((NP,), jnp.float32),
        ],
    )


def _make_post_kernel(N, NP, D, BN):
    H = D // 2

    def body(agg_ref, y2_ref, dis_ref, x_ref, b_ref, lw_ref, lb_ref, o_ref):
        agg = jnp.concatenate([agg_ref[0], agg_ref[1]], axis=1)
        y = jnp.concatenate([y2_ref[0], y2_ref[1]], axis=1)
        dis = dis_ref[...]
        h = (agg + y) * dis[:, None] + b_ref[...][None, :]
        mu = jnp.mean(h, axis=1, keepdims=True)
        d = h - mu
        var = jnp.mean(d * d, axis=1, keepdims=True)
        h = d * lax.rsqrt(var + 1e-5) * lw_ref[...][None, :] + lb_ref[...][None, :]
        h = jnp.maximum(h, 0.0)
        o_ref[...] = h + x_ref[...]

    return pl.pallas_call(
        body,
        grid=(NP // BN,),
        in_specs=[
            pl.BlockSpec((2, BN, H), lambda i: (0, i, 0)),
            pl.BlockSpec((2, BN, H), lambda i: (0, i, 0)),
            pl.BlockSpec((BN,), lambda i: (i,)),
            pl.BlockSpec((BN, D), lambda i: (i, 0)),
            pl.BlockSpec((D,), lambda i: (0,)),
            pl.BlockSpec((D,), lambda i: (0,)),
            pl.BlockSpec((D,), lambda i: (0,)),
        ],
        out_specs=pl.BlockSpec((BN, D), lambda i: (i, 0)),
        out_shape=jax.ShapeDtypeStruct((N, D), jnp.float32),
    )


def kernel(x, edge_index, W, b, ln_w, ln_b):
    N, D = x.shape
    E = edge_index.shape[1]
    H = D // 2
    NP = ((N + 2047) // 2048) * 2048
    EP = ((E + 4095) // 4096) * 4096
    BN = 512

    row = edge_index[0]
    col = edge_index[1]
    pad = EP - E
    if pad:
        fill = (N + (jnp.arange(pad, dtype=jnp.int32) % (NP - N))).astype(jnp.int32)
        row = jnp.concatenate([row, fill])
        col = jnp.concatenate([col, fill])
    row2 = jnp.concatenate([row, row + NP])

    col2d = col.reshape(EP // CH, CH)
    rowc = row2.reshape(2, EP // CH, CH)
    colc = jnp.broadcast_to(col2d[None], (2, EP // CH, CH))
    rc = jnp.stack([rowc, colc], axis=2).reshape(4 * (EP // CH), CH)
    degs = _make_deg_kernel(NP, EP)(col2d)
    y2, dis = _make_pre_kernel(N, NP, D, BN)(x, W, degs.reshape(2, NP))
    agg = _make_agg_kernel(NP, EP, H)(y2.reshape(2 * NP, H), rc)
    return _make_post_kernel(N, NP, D, BN)(
        agg.reshape(2, NP, H), y2, dis, x, b, ln_w, ln_b
    )


# speedup vs baseline: 21.5666x; 1.0169x over previous
import functools

import jax
import jax.numpy as jnp
from jax import lax
from jax.experimental import pallas as pl
from jax.experimental.pallas import tpu as pltpu
from jax.experimental.pallas import tpu_sc as plsc

CH = 128


def _sc_mesh():
    return plsc.VectorSubcoreMesh(core_axis_name="c", subcore_axis_name="s")


def _make_deg_kernel(NP, EP):
    n_per_tile = EP // 32
    nch = n_per_tile // CH
    slc = NP // 16

    @functools.partial(
        pl.kernel,
        mesh=_sc_mesh(),
        out_type=jax.ShapeDtypeStruct((2 * NP,), jnp.float32),
        scratch_types=[
            pltpu.VMEM_SHARED((NP,), jnp.float32),
            pltpu.VMEM((CH,), jnp.float32),
            pltpu.VMEM((nch, CH), jnp.int32),
            pltpu.VMEM((slc,), jnp.float32),
            pltpu.SemaphoreType.DMA,
        ],
    )
    def deg_k(col2d_hbm, deg_out, deg_sh, ones_v, cidx_all, zb_v, sem):
        c = lax.axis_index("c")
        s = lax.axis_index("s")
        tch = c * (EP // 2 // CH) + s * nch

        def fill_ones(i, _):
            ones_v[pl.ds(i * 16, 16)] = jnp.full((16,), 1.0, jnp.float32)
            return 0

        lax.fori_loop(0, CH // 16, fill_ones, 0)

        def fill_z(i, _):
            zb_v[pl.ds(i * 16, 16)] = jnp.zeros((16,), jnp.float32)
            return 0

        lax.fori_loop(0, slc // 16, fill_z, 0)
        pltpu.sync_copy(col2d_hbm.at[pl.ds(tch, nch)], cidx_all)
        pltpu.sync_copy(zb_v, deg_sh.at[pl.ds(s * slc, slc)])
        plsc.subcore_barrier()

        def fire(j, _):
            pltpu.async_copy(ones_v, deg_sh.at[cidx_all.at[j]], sem, add=True)
            return 0

        lax.fori_loop(0, nch, fire, 0)

        def drain(j, _):
            pltpu.make_async_copy(ones_v, deg_sh.at[cidx_all.at[j]], sem).wait()
            return 0

        lax.fori_loop(0, nch, drain, 0)
        plsc.subcore_barrier()
        pltpu.sync_copy(
            deg_sh.at[pl.ds(s * slc, slc)],
            deg_out.at[pl.ds(c * NP + s * slc, slc)],
        )

    return deg_k


def _make_agg_kernel(NP, EP, H):
    n_per_tile = EP // 16
    nch = n_per_tile // CH
    slc = NP // 16

    assert nch % 4 == 0

    @functools.partial(
        pl.kernel,
        mesh=_sc_mesh(),
        out_type=jax.ShapeDtypeStruct((2 * NP, H), jnp.float32),
        scratch_types=[
            pltpu.VMEM_SHARED((NP, H), jnp.float32),
            pltpu.VMEM((CH, H), jnp.float32),
            pltpu.VMEM((CH, H), jnp.float32),
            pltpu.VMEM((2, CH), jnp.int32),
            pltpu.VMEM((2, CH), jnp.int32),
            pltpu.VMEM((2, CH), jnp.int32),
            pltpu.VMEM((2, CH), jnp.int32),
            pltpu.SemaphoreType.DMA,
            pltpu.SemaphoreType.DMA,
            pltpu.SemaphoreType.DMA,
            pltpu.SemaphoreType.DMA,
            pltpu.SemaphoreType.DMA,
            pltpu.SemaphoreType.DMA,
            pltpu.SemaphoreType.DMA,
            pltpu.SemaphoreType.DMA,
        ],
    )
    def agg_k(y_hbm, rc_hbm, out_hbm, agg_sh, bufa, bufb, r0, r1, r2, r3,
              semga, semgb, semsa, semsb, semr0, semr1, semr2, semr3):
        c = lax.axis_index("c")
        s = lax.axis_index("s")
        tbase = 2 * (c * (EP // CH) + s * nch)

        def rstart(j, rc, semr):
            pltpu.async_copy(rc_hbm.at[pl.ds(tbase + 2 * j, 2)], rc, semr)

        def rwait(j, rc, semr):
            pltpu.make_async_copy(
                rc_hbm.at[pl.ds(tbase + 2 * j, 2)], rc, semr
            ).wait()

        def gstart(rc, buf, sem):
            pltpu.async_copy(y_hbm.at[rc.at[0]], buf, sem)

        def gwait(rc, buf, sem):
            pltpu.make_async_copy(y_hbm.at[rc.at[0]], buf, sem).wait()

        def sstart(buf, rc, sem):
            pltpu.async_copy(buf, agg_sh.at[rc.at[1]], sem, add=True)

        def swait(buf, rc, sem):
            pltpu.make_async_copy(buf, agg_sh.at[rc.at[1]], sem).wait()

        rstart(0, r0, semr0)
        rstart(1, r1, semr1)
        rstart(2, r2, semr2)

        pltpu.sync_copy(
            y_hbm.at[pl.ds(c * NP + s * slc, slc)],
            agg_sh.at[pl.ds(s * slc, slc)],
        )
        rwait(0, r0, semr0)
        gstart(r0, bufa, semga)
        plsc.subcore_barrier()

        def sub(j, bj, sgj, ssj, bp, ssp, sgp, rcj, rcn, rcp, semrn, semrp,
                first, guard_pref, guard_g):
            gwait(rcj, bj, sgj)
            sstart(bj, rcj, ssj)

            def after_prev():
                swait(bp, rcp, ssp)

            if first:
                @pl.when(j > 0)
                def _():
                    after_prev()
            else:
                after_prev()

            if guard_pref:
                @pl.when(j + 3 < nch)
                def _():
                    rstart(j + 3, rcp, semrp)
            else:
                rstart(j + 3, rcp, semrp)

            def next_gather():
                rwait(j + 1, rcn, semrn)
                gstart(rcn, bp, sgp)

            if guard_g:
                @pl.when(j + 1 < nch)
                def _():
                    next_gather()
            else:
                next_gather()

        def step(i, _):
            j0 = 4 * i
            sub(j0, bufa, semga, semsa, bufb, semsb, semgb,
                r0, r1, r3, semr1, semr3, True, False, False)
            sub(j0 + 1, bufb, semgb, semsb, bufa, semsa, semga,
                r1, r2, r0, semr2, semr0, False, True, False)
            sub(j0 + 2, bufa, semga, semsa, bufb, semsb, semgb,
                r2, r3, r1, semr3, semr1, False, True, False)
            sub(j0 + 3, bufb, semgb, semsb, bufa, semsa, semga,
                r3, r0, r2, semr0, semr2, False, True, True)
            return 0

        lax.fori_loop(0, nch // 4, step, 0)
        swait(bufb, r3, semsb)
        plsc.subcore_barrier()
        pltpu.sync_copy(
            agg_sh.at[pl.ds(s * slc, slc)],
            out_hbm.at[pl.ds(c * NP + s * slc, slc)],
        )

    return agg_k


def _make_pre_kernel(N, NP, D, BN):
    H = D // 2

    def body(x_ref, w_ref, degs_ref, y2_ref, dis_ref):
        deg = degs_ref[0] + degs_ref[1] + 1.0
        dis = lax.rsqrt(deg)
        xw = jnp.dot(x_ref[...], w_ref[...], preferred_element_type=jnp.float32)
        y = xw * dis[:, None]
        y2_ref[0] = y[:, :H]
        y2_ref[1] = y[:, H:]
        dis_ref[...] = dis

    return pl.pallas_call(
        body,
        grid=(NP // BN,),
        in_specs=[
            pl.BlockSpec((BN, D), lambda i: (i, 0)),
            pl.BlockSpec((D, D), lambda i: (0, 0)),
            pl.BlockSpec((2, BN), lambda i: (0, i)),
        ],
        out_specs=[
            pl.BlockSpec((2, BN, H), lambda i: (0, i, 0)),
            pl.BlockSpec((BN,), lambda i: (i,)),
        ],
        out_shape=[
            jax.ShapeDtypeStruct((2, NP, H), jnp.float32),
            jax.ShapeDtypeStruct((NP,), jnp.float32),
        ],
    )


def _make_post_kernel(N, NP, D, BN):
    H = D // 2

    def body(agg_ref, dis_ref, x_ref, b_ref, lw_ref, lb_ref, o_ref):
        agg = jnp.concatenate([agg_ref[0], agg_ref[1]], axis=1)
        dis = dis_ref[...]
        h = agg * dis[:, None] + b_ref[...][None, :]
        mu = jnp.mean(h, axis=1, keepdims=True)
        d = h - mu
        var = jnp.mean(d * d, axis=1, keepdims=True)
        h = d * lax.rsqrt(var + 1e-5) * lw_ref[...][None, :] + lb_ref[...][None, :]
        h = jnp.maximum(h, 0.0)
        o_ref[...] = h + x_ref[...]

    return pl.pallas_call(
        body,
        grid=(NP // BN,),
        in_specs=[
            pl.BlockSpec((2, BN, H), lambda i: (0, i, 0)),
            pl.BlockSpec((BN,), lambda i: (i,)),
            pl.BlockSpec((BN, D), lambda i: (i, 0)),
            pl.BlockSpec((D,), lambda i: (0,)),
            pl.BlockSpec((D,), lambda i: (0,)),
            pl.BlockSpec((D,), lambda i: (0,)),
        ],
        out_specs=pl.BlockSpec((BN, D), lambda i: (i, 0)),
        out_shape=jax.ShapeDtypeStruct((N, D), jnp.float32),
    )


def kernel(x, edge_index, W, b, ln_w, ln_b):
    N, D = x.shape
    E = edge_index.shape[1]
    H = D // 2
    NP = ((N + 2047) // 2048) * 2048
    EP = ((E + 4095) // 4096) * 4096
    BN = 512

    row = edge_index[0]
    col = edge_index[1]
    pad = EP - E
    if pad:
        fill = (N + (jnp.arange(pad, dtype=jnp.int32) % (NP - N))).astype(jnp.int32)
        row = jnp.concatenate([row, fill])
        col = jnp.concatenate([col, fill])
    row2 = jnp.concatenate([row, row + NP])

    col2d = col.reshape(EP // CH, CH)
    rowc = row2.reshape(2, EP // CH, CH)
    colc = jnp.broadcast_to(col2d[None], (2, EP // CH, CH))
    rc = jnp.stack([rowc, colc], axis=2).reshape(4 * (EP // CH), CH)
    degs = _make_deg_kernel(NP, EP)(col2d)
    y2, dis = _make_pre_kernel(N, NP, D, BN)(x, W, degs.reshape(2, NP))
    agg = _make_agg_kernel(NP, EP, H)(y2.reshape(2 * NP, H), rc)
    return _make_post_kernel(N, NP, D, BN)(
        agg.reshape(2, NP, H), dis, x, b, ln_w, ln_b
    )


# speedup vs baseline: 22.6828x; 1.0518x over previous
import functools

import jax
import jax.numpy as jnp
from jax import lax
from jax.experimental import pallas as pl
from jax.experimental.pallas import tpu as pltpu
from jax.experimental.pallas import tpu_sc as plsc

CH = 128


def _sc_mesh():
    return plsc.VectorSubcoreMesh(core_axis_name="c", subcore_axis_name="s")


def _make_deg_kernel(NP, EP):
    n_per_tile = EP // 32
    nch = n_per_tile // CH
    slc = NP // 16

    @functools.partial(
        pl.kernel,
        mesh=_sc_mesh(),
        out_type=jax.ShapeDtypeStruct((2 * NP,), jnp.float32),
        scratch_types=[
            pltpu.VMEM_SHARED((NP,), jnp.float32),
            pltpu.VMEM((CH,), jnp.float32),
            pltpu.VMEM((nch, CH), jnp.int32),
            pltpu.VMEM((slc,), jnp.float32),
            pltpu.SemaphoreType.DMA,
        ],
    )
    def deg_k(col2d_hbm, deg_out, deg_sh, ones_v, cidx_all, zb_v, sem):
        c = lax.axis_index("c")
        s = lax.axis_index("s")
        tch = c * (EP // 2 // CH) + s * nch

        def fill_ones(i, _):
            ones_v[pl.ds(i * 16, 16)] = jnp.full((16,), 1.0, jnp.float32)
            return 0

        lax.fori_loop(0, CH // 16, fill_ones, 0)

        def fill_z(i, _):
            zb_v[pl.ds(i * 16, 16)] = jnp.zeros((16,), jnp.float32)
            return 0

        lax.fori_loop(0, slc // 16, fill_z, 0)
        pltpu.sync_copy(col2d_hbm.at[pl.ds(tch, nch)], cidx_all)
        pltpu.sync_copy(zb_v, deg_sh.at[pl.ds(s * slc, slc)])
        plsc.subcore_barrier()

        def fire(j, _):
            pltpu.async_copy(ones_v, deg_sh.at[cidx_all.at[j]], sem, add=True)
            return 0

        lax.fori_loop(0, nch, fire, 0)

        def drain(j, _):
            pltpu.make_async_copy(ones_v, deg_sh.at[cidx_all.at[j]], sem).wait()
            return 0

        lax.fori_loop(0, nch, drain, 0)
        plsc.subcore_barrier()
        pltpu.sync_copy(
            deg_sh.at[pl.ds(s * slc, slc)],
            deg_out.at[pl.ds(c * NP + s * slc, slc)],
        )

    return deg_k


def _make_agg_kernel(NP, EP, H):
    n_per_tile = EP // 16
    nch = n_per_tile // CH
    slc = NP // 16

    assert nch % 4 == 0

    @functools.partial(
        pl.kernel,
        mesh=_sc_mesh(),
        out_type=jax.ShapeDtypeStruct((2 * NP, H), jnp.float32),
        scratch_types=[
            pltpu.VMEM_SHARED((NP, H), jnp.float32),
            pltpu.VMEM((CH, H), jnp.float32),
            pltpu.VMEM((CH, H), jnp.float32),
            pltpu.VMEM((2, CH), jnp.int32),
            pltpu.VMEM((2, CH), jnp.int32),
            pltpu.VMEM((2, CH), jnp.int32),
            pltpu.VMEM((2, CH), jnp.int32),
            pltpu.SemaphoreType.DMA,
            pltpu.SemaphoreType.DMA,
            pltpu.SemaphoreType.DMA,
            pltpu.SemaphoreType.DMA,
            pltpu.SemaphoreType.DMA,
            pltpu.SemaphoreType.DMA,
            pltpu.SemaphoreType.DMA,
            pltpu.SemaphoreType.DMA,
        ],
    )
    def agg_k(y_hbm, rc_hbm, out_hbm, agg_sh, bufa, bufb, r0, r1, r2, r3,
              semga, semgb, semsa, semsb, semr0, semr1, semr2, semr3):
        c = lax.axis_index("c")
        s = lax.axis_index("s")
        tbase = 2 * s * nch
        roff = c * NP

        def rstart(j, rc, semr):
            pltpu.async_copy(rc_hbm.at[pl.ds(tbase + 2 * j, 2)], rc, semr)

        def rwait(j, rc, semr):
            pltpu.make_async_copy(
                rc_hbm.at[pl.ds(tbase + 2 * j, 2)], rc, semr
            ).wait()

            def adj(k, _):
                rc[0, pl.ds(k * 16, 16)] = rc[0, pl.ds(k * 16, 16)] + roff
                return 0

            lax.fori_loop(0, CH // 16, adj, 0)

        def gstart(rc, buf, sem):
            pltpu.async_copy(y_hbm.at[rc.at[0]], buf, sem)

        def gwait(rc, buf, sem):
            pltpu.make_async_copy(y_hbm.at[rc.at[0]], buf, sem).wait()

        def sstart(buf, rc, sem):
            pltpu.async_copy(buf, agg_sh.at[rc.at[1]], sem, add=True)

        def swait(buf, rc, sem):
            pltpu.make_async_copy(buf, agg_sh.at[rc.at[1]], sem).wait()

        rstart(0, r0, semr0)
        rstart(1, r1, semr1)
        rstart(2, r2, semr2)

        pltpu.sync_copy(
            y_hbm.at[pl.ds(c * NP + s * slc, slc)],
            agg_sh.at[pl.ds(s * slc, slc)],
        )
        rwait(0, r0, semr0)
        gstart(r0, bufa, semga)
        plsc.subcore_barrier()

        def sub(j, bj, sgj, ssj, bp, ssp, sgp, rcj, rcn, rcp, semrn, semrp,
                first, guard_pref, guard_g):
            gwait(rcj, bj, sgj)
            sstart(bj, rcj, ssj)

            def after_prev():
                swait(bp, rcp, ssp)

            if first:
                @pl.when(j > 0)
                def _():
                    after_prev()
            else:
                after_prev()

            if guard_pref:
                @pl.when(j + 3 < nch)
                def _():
                    rstart(j + 3, rcp, semrp)
            else:
                rstart(j + 3, rcp, semrp)

            def next_gather():
                rwait(j + 1, rcn, semrn)
                gstart(rcn, bp, sgp)

            if guard_g:
                @pl.when(j + 1 < nch)
                def _():
                    next_gather()
            else:
                next_gather()

        def step(i, _):
            j0 = 4 * i
            sub(j0, bufa, semga, semsa, bufb, semsb, semgb,
                r0, r1, r3, semr1, semr3, True, False, False)
            sub(j0 + 1, bufb, semgb, semsb, bufa, semsa, semga,
                r1, r2, r0, semr2, semr0, False, True, False)
            sub(j0 + 2, bufa, semga, semsa, bufb, semsb, semgb,
                r2, r3, r1, semr3, semr1, False, True, False)
            sub(j0 + 3, bufb, semgb, semsb, bufa, semsa, semga,
                r3, r0, r2, semr0, semr2, False, True, True)
            return 0

        lax.fori_loop(0, nch // 4, step, 0)
        swait(bufb, r3, semsb)
        plsc.subcore_barrier()
        pltpu.sync_copy(
            agg_sh.at[pl.ds(s * slc, slc)],
            out_hbm.at[pl.ds(c * NP + s * slc, slc)],
        )

    return agg_k


def _make_pre_kernel(N, NP, D, BN):
    H = D // 2

    def body(x_ref, w_ref, degs_ref, y2_ref, dis_ref):
        deg = degs_ref[0] + degs_ref[1] + 1.0
        dis = lax.rsqrt(deg)
        xw = jnp.dot(x_ref[...], w_ref[...], preferred_element_type=jnp.float32)
        y = xw * dis[:, None]
        y2_ref[0] = y[:, :H]
        y2_ref[1] = y[:, H:]
        dis_ref[...] = dis

    return pl.pallas_call(
        body,
        grid=(NP // BN,),
        in_specs=[
            pl.BlockSpec((BN, D), lambda i: (i, 0)),
            pl.BlockSpec((D, D), lambda i: (0, 0)),
            pl.BlockSpec((2, BN), lambda i: (0, i)),
        ],
        out_specs=[
            pl.BlockSpec((2, BN, H), lambda i: (0, i, 0)),
            pl.BlockSpec((BN,), lambda i: (i,)),
        ],
        out_shape=[
            jax.ShapeDtypeStruct((2, NP, H), jnp.float32),
            jax.ShapeDtypeStruct((NP,), jnp.float32),
        ],
    )


def _make_post_kernel(N, NP, D, BN):
    H = D // 2

    def body(agg_ref, dis_ref, x_ref, b_ref, lw_ref, lb_ref, o_ref):
        agg = jnp.concatenate([agg_ref[0], agg_ref[1]], axis=1)
        dis = dis_ref[...]
        h = agg * dis[:, None] + b_ref[...][None, :]
        mu = jnp.mean(h, axis=1, keepdims=True)
        d = h - mu
        var = jnp.mean(d * d, axis=1, keepdims=True)
        h = d * lax.rsqrt(var + 1e-5) * lw_ref[...][None, :] + lb_ref[...][None, :]
        h = jnp.maximum(h, 0.0)
        o_ref[...] = h + x_ref[...]

    return pl.pallas_call(
        body,
        grid=(NP // BN,),
        in_specs=[
            pl.BlockSpec((2, BN, H), lambda i: (0, i, 0)),
            pl.BlockSpec((BN,), lambda i: (i,)),
            pl.BlockSpec((BN, D), lambda i: (i, 0)),
            pl.BlockSpec((D,), lambda i: (0,)),
            pl.BlockSpec((D,), lambda i: (0,)),
            pl.BlockSpec((D,), lambda i: (0,)),
        ],
        out_specs=pl.BlockSpec((BN, D), lambda i: (i, 0)),
        out_shape=jax.ShapeDtypeStruct((N, D), jnp.float32),
    )


def kernel(x, edge_index, W, b, ln_w, ln_b):
    N, D = x.shape
    E = edge_index.shape[1]
    H = D // 2
    NP = ((N + 2047) // 2048) * 2048
    EP = ((E + 4095) // 4096) * 4096
    BN = 1024

    row = edge_index[0]
    col = edge_index[1]
    pad = EP - E
    if pad:
        fill = (N + (jnp.arange(pad, dtype=jnp.int32) % (NP - N))).astype(jnp.int32)
        row = jnp.concatenate([row, fill])
        col = jnp.concatenate([col, fill])
    col2d = col.reshape(EP // CH, CH)
    rc = jnp.stack(
        [row.reshape(EP // CH, CH), col2d], axis=1
    ).reshape(2 * (EP // CH), CH)
    degs = _make_deg_kernel(NP, EP)(col2d)
    y2, dis = _make_pre_kernel(N, NP, D, BN)(x, W, degs.reshape(2, NP))
    agg = _make_agg_kernel(NP, EP, H)(y2.reshape(2 * NP, H), rc)
    return _make_post_kernel(N, NP, D, BN)(
        agg.reshape(2, NP, H), dis, x, b, ln_w, ln_b
    )


# speedup vs baseline: 22.7340x; 1.0023x over previous
import functools

import jax
import jax.numpy as jnp
from jax import lax
from jax.experimental import pallas as pl
from jax.experimental.pallas import tpu as pltpu
from jax.experimental.pallas import tpu_sc as plsc

CH = 128


def _sc_mesh():
    return plsc.VectorSubcoreMesh(core_axis_name="c", subcore_axis_name="s")


def _make_deg_kernel(NP, EP):
    n_per_tile = EP // 32
    nch = n_per_tile // CH
    slc = NP // 16

    @functools.partial(
        pl.kernel,
        mesh=_sc_mesh(),
        out_type=jax.ShapeDtypeStruct((2 * NP,), jnp.float32),
        scratch_types=[
            pltpu.VMEM_SHARED((NP,), jnp.float32),
            pltpu.VMEM((CH,), jnp.float32),
            pltpu.VMEM((nch, CH), jnp.int32),
            pltpu.VMEM((slc,), jnp.float32),
            pltpu.SemaphoreType.DMA,
        ],
    )
    def deg_k(col2d_hbm, deg_out, deg_sh, ones_v, cidx_all, zb_v, sem):
        c = lax.axis_index("c")
        s = lax.axis_index("s")
        tch = c * (EP // 2 // CH) + s * nch

        def fill_ones(i, _):
            ones_v[pl.ds(i * 16, 16)] = jnp.full((16,), 1.0, jnp.float32)
            return 0

        lax.fori_loop(0, CH // 16, fill_ones, 0)

        def fill_z(i, _):
            zb_v[pl.ds(i * 16, 16)] = jnp.zeros((16,), jnp.float32)
            return 0

        lax.fori_loop(0, slc // 16, fill_z, 0)
        pltpu.sync_copy(col2d_hbm.at[pl.ds(tch, nch)], cidx_all)
        pltpu.sync_copy(zb_v, deg_sh.at[pl.ds(s * slc, slc)])
        plsc.subcore_barrier()

        def fire(j, _):
            pltpu.async_copy(ones_v, deg_sh.at[cidx_all.at[j]], sem, add=True)
            return 0

        lax.fori_loop(0, nch, fire, 0)

        def drain(j, _):
            pltpu.make_async_copy(ones_v, deg_sh.at[cidx_all.at[j]], sem).wait()
            return 0

        lax.fori_loop(0, nch, drain, 0)
        plsc.subcore_barrier()
        pltpu.sync_copy(
            deg_sh.at[pl.ds(s * slc, slc)],
            deg_out.at[pl.ds(c * NP + s * slc, slc)],
        )

    return deg_k


def _make_agg_kernel(NP, EP, H):
    n_per_tile = EP // 16
    nch = n_per_tile // CH
    slc = NP // 16

    assert nch % 4 == 0

    @functools.partial(
        pl.kernel,
        mesh=_sc_mesh(),
        out_type=jax.ShapeDtypeStruct((2 * NP, H), jnp.float32),
        scratch_types=[
            pltpu.VMEM_SHARED((NP, H), jnp.float32),
            pltpu.VMEM((CH, H), jnp.float32),
            pltpu.VMEM((CH, H), jnp.float32),
            pltpu.VMEM((2, CH), jnp.int32),
            pltpu.VMEM((2, CH), jnp.int32),
            pltpu.VMEM((2, CH), jnp.int32),
            pltpu.VMEM((2, CH), jnp.int32),
            pltpu.SemaphoreType.DMA,
            pltpu.SemaphoreType.DMA,
            pltpu.SemaphoreType.DMA,
            pltpu.SemaphoreType.DMA,
            pltpu.SemaphoreType.DMA,
            pltpu.SemaphoreType.DMA,
            pltpu.SemaphoreType.DMA,
            pltpu.SemaphoreType.DMA,
        ],
    )
    def agg_k(y_hbm, rc_hbm, out_hbm, agg_sh, bufa, bufb, r0, r1, r2, r3,
              semga, semgb, semsa, semsb, semr0, semr1, semr2, semr3):
        c = lax.axis_index("c")
        s = lax.axis_index("s")
        tbase = 2 * s * nch
        roff = c * NP

        def rstart(j, rc, semr):
            pltpu.async_copy(rc_hbm.at[pl.ds(tbase + 2 * j, 2)], rc, semr)

        def rwait(j, rc, semr):
            pltpu.make_async_copy(
                rc_hbm.at[pl.ds(tbase + 2 * j, 2)], rc, semr
            ).wait()

            def adj(k, _):
                rc[0, pl.ds(k * 16, 16)] = rc[0, pl.ds(k * 16, 16)] + roff
                return 0

            lax.fori_loop(0, CH // 16, adj, 0)

        def gstart(rc, buf, sem):
            pltpu.async_copy(y_hbm.at[rc.at[0]], buf, sem)

        def gwait(rc, buf, sem):
            pltpu.make_async_copy(y_hbm.at[rc.at[0]], buf, sem).wait()

        def sstart(buf, rc, sem):
            pltpu.async_copy(buf, agg_sh.at[rc.at[1]], sem, add=True)

        def swait(buf, rc, sem):
            pltpu.make_async_copy(buf, agg_sh.at[rc.at[1]], sem).wait()

        rstart(0, r0, semr0)
        rstart(1, r1, semr1)
        rstart(2, r2, semr2)

        pltpu.sync_copy(
            y_hbm.at[pl.ds(c * NP + s * slc, slc)],
            agg_sh.at[pl.ds(s * slc, slc)],
        )
        rwait(0, r0, semr0)
        gstart(r0, bufa, semga)
        plsc.subcore_barrier()

        def sub(j, bj, sgj, ssj, bp, ssp, sgp, rcj, rcn, rcp, semrn, semrp,
                first, guard_pref, guard_g):
            gwait(rcj, bj, sgj)
            sstart(bj, rcj, ssj)

            def after_prev():
                swait(bp, rcp, ssp)

            if first:
                @pl.when(j > 0)
                def _():
                    after_prev()
            else:
                after_prev()

            if guard_pref:
                @pl.when(j + 3 < nch)
                def _():
                    rstart(j + 3, rcp, semrp)
            else:
                rstart(j + 3, rcp, semrp)

            def next_gather():
                rwait(j + 1, rcn, semrn)
                gstart(rcn, bp, sgp)

            if guard_g:
                @pl.when(j + 1 < nch)
                def _():
                    next_gather()
            else:
                next_gather()

        def step(i, _):
            j0 = 4 * i
            sub(j0, bufa, semga, semsa, bufb, semsb, semgb,
                r0, r1, r3, semr1, semr3, True, False, False)
            sub(j0 + 1, bufb, semgb, semsb, bufa, semsa, semga,
                r1, r2, r0, semr2, semr0, False, True, False)
            sub(j0 + 2, bufa, semga, semsa, bufb, semsb, semgb,
                r2, r3, r1, semr3, semr1, False, True, False)
            sub(j0 + 3, bufb, semgb, semsb, bufa, semsa, semga,
                r3, r0, r2, semr0, semr2, False, True, True)
            return 0

        lax.fori_loop(0, nch // 4, step, 0)
        swait(bufb, r3, semsb)
        plsc.subcore_barrier()
        pltpu.sync_copy(
            agg_sh.at[pl.ds(s * slc, slc)],
            out_hbm.at[pl.ds(c * NP + s * slc, slc)],
        )

    return agg_k


def _make_pre_kernel(N, NP, D, BN):
    H = D // 2

    def body(x_ref, w_ref, degs_ref, y2_ref, dis_ref):
        deg = degs_ref[0] + degs_ref[1] + 1.0
        dis = lax.rsqrt(deg)
        xw = jnp.dot(x_ref[...], w_ref[...], preferred_element_type=jnp.float32)
        y = xw * dis[:, None]
        y2_ref[0] = y[:, :H]
        y2_ref[1] = y[:, H:]
        dis_ref[...] = dis

    return pl.pallas_call(
        body,
        grid=(NP // BN,),
        in_specs=[
            pl.BlockSpec((BN, D), lambda i: (i, 0)),
            pl.BlockSpec((D, D), lambda i: (0, 0)),
            pl.BlockSpec((2, BN), lambda i: (0, i)),
        ],
        out_specs=[
            pl.BlockSpec((2, BN, H), lambda i: (0, i, 0)),
            pl.BlockSpec((BN,), lambda i: (i,)),
        ],
        out_shape=[
            jax.ShapeDtypeStruct((2, NP, H), jnp.float32),
            jax.ShapeDtypeStruct((NP,), jnp.float32),
        ],
    )


def _make_post_kernel(N, NP, D, BN):
    H = D // 2

    def body(agg_ref, dis_ref, x_ref, b_ref, lw_ref, lb_ref, o_ref):
        agg = jnp.concatenate([agg_ref[0], agg_ref[1]], axis=1)
        dis = dis_ref[...]
        h = agg * dis[:, None] + b_ref[...][None, :]
        mu = jnp.mean(h, axis=1, keepdims=True)
        d = h - mu
        var = jnp.mean(d * d, axis=1, keepdims=True)
        h = d * lax.rsqrt(var + 1e-5) * lw_ref[...][None, :] + lb_ref[...][None, :]
        h = jnp.maximum(h, 0.0)
        o_ref[...] = h + x_ref[...]

    return pl.pallas_call(
        body,
        grid=(NP // BN,),
        in_specs=[
            pl.BlockSpec((2, BN, H), lambda i: (0, i, 0)),
            pl.BlockSpec((BN,), lambda i: (i,)),
            pl.BlockSpec((BN, D), lambda i: (i, 0)),
            pl.BlockSpec((D,), lambda i: (0,)),
            pl.BlockSpec((D,), lambda i: (0,)),
            pl.BlockSpec((D,), lambda i: (0,)),
        ],
        out_specs=pl.BlockSpec((BN, D), lambda i: (i, 0)),
        out_shape=jax.ShapeDtypeStruct((N, D), jnp.float32),
    )


def kernel(x, edge_index, W, b, ln_w, ln_b):
    N, D = x.shape
    E = edge_index.shape[1]
    H = D // 2
    NP = ((N + 2048) // 2048) * 2048
    EP = ((E + 4095) // 4096) * 4096
    BN = 1024

    row = edge_index[0]
    col = edge_index[1]
    pad = EP - E
    if pad:
        fill = (N + (jnp.arange(pad, dtype=jnp.int32) % (NP - N))).astype(jnp.int32)
        row = jnp.concatenate([row, fill])
        col = jnp.concatenate([col, fill])
    col2d = col.reshape(EP // CH, CH)
    rc = jnp.stack(
        [row.reshape(EP // CH, CH), col2d], axis=1
    ).reshape(2 * (EP // CH), CH)
    degs = _make_deg_kernel(NP, EP)(col2d)
    y2, dis = _make_pre_kernel(N, NP, D, BN)(x, W, degs.reshape(2, NP))
    agg = _make_agg_kernel(NP, EP, H)(y2.reshape(2 * NP, H), rc)
    return _make_post_kernel(N, NP, D, BN)(
        agg.reshape(2, NP, H), dis, x, b, ln_w, ln_b
    )
